# baseline (speedup 1.0000x reference)
import functools

import jax
import jax.numpy as jnp
from jax import lax
from jax.experimental import pallas as pl
from jax.experimental.pallas import tpu as pltpu
from jax.experimental.pallas import tpu_sc as plsc

N = 1_000_000
K = 100_000
L = 16
NT = 16
CH = 62_720
NV = CH // L
HV = NV // 2
H = HV * L
U = 8
TAIL = NT * CH - N
LAST_LOAD = CH - TAIL
LAST_STORE = H - TAIL
BV = 20
NBC = NV // BV
NB = HV // BV


def _iota():
    return lax.iota(jnp.int32, L)


def _sc_body(x_hbm, out_hbm, data_v, mask_v, hist_v, merged_v, eq_v, blk_v,
             blka_v, blkb_v, shist_s, seq_s):
    c = lax.axis_index("c")
    s = lax.axis_index("s")
    lane = _iota()
    ones_i = jnp.ones((L,), jnp.int32)
    zeros_i = jnp.zeros((L,), jnp.int32)
    u32_ff = jnp.uint32(0xFF)

    def lane0(v):
        return jnp.sum(jnp.where(lane == 0, v, 0))

    def lane1(v):
        return jnp.sum(jnp.where(lane == 1, v, 0))

    with jax.named_scope("ph_load"):
        @pl.when(s != NT - 1)
        def _():
            pltpu.sync_copy(x_hbm.at[pl.ds(s * CH, CH)], data_v)

        @pl.when(s == NT - 1)
        def _():
            pltpu.sync_copy(x_hbm.at[pl.ds(s * CH, LAST_LOAD)],
                            data_v.at[pl.ds(0, LAST_LOAD)])
            ninf = jnp.full((L,), -jnp.inf, jnp.float32)
            def fill(i, _):
                data_v[pl.ds(LAST_LOAD + i * L, L)] = ninf
                return 0
            lax.fori_loop(0, TAIL // L, fill, 0)

    def zero_hist():
        def z(i, _):
            base = i * (U * L)
            for j in range(U):
                hist_v[pl.ds(base + j * L, L)] = zeros_i
            return 0
        lax.fori_loop(0, 256 // U, z, 0)

    def lane_merge():
        def m(g, _):
            base = (g * L + lane) * L
            acc = zeros_i
            for l in range(L):
                acc = acc + plsc.load_gather(hist_v, [base + l])
            merged_v[pl.ds(g * L, L)] = acc
            return 0
        lax.fori_loop(0, L, m, 0)

    def scan_level(k_rem):
        def scan_g(gi, carry):
            acc, found, bin_f, above_f = carry
            gd = 15 - gi
            def gsum(j, a):
                return a + hist_v[pl.ds(j * 256 + gd * L, L)]
            tot = lax.fori_loop(0, NT, gsum, zeros_i)
            tr = jnp.flip(tot, 0)
            cs = plsc.cumsum(tr)
            sfx = cs + acc
            hit = sfx >= k_rem
            anyh = jnp.any(hit)
            p = jnp.max(plsc.all_reduce_ffs(hit))
            onehot = lane == p
            sfx_at = jnp.sum(jnp.where(onehot, sfx, 0))
            h_at = jnp.sum(jnp.where(onehot, tr, 0))
            new = anyh & (found == 0)
            bin_f = jnp.where(new, gd * L + 15 - p, bin_f)
            above_f = jnp.where(new, sfx_at - h_at, above_f)
            found = jnp.where(anyh, 1, found)
            acc = acc + jnp.max(cs)
            return acc, found, bin_f, above_f
        _, _, bin_f, above_f = lax.fori_loop(
            0, L, scan_g,
            (jnp.int32(0), jnp.int32(0), jnp.int32(0), jnp.int32(0)))
        return bin_f, above_f

    def merge_and_scan(k_rem):
        pltpu.sync_copy(merged_v, shist_s.at[pl.ds(s * 256, 256)])
        plsc.subcore_barrier()

        @pl.when(s == 0)
        def _():
            pltpu.sync_copy(shist_s, hist_v)
            bin_f, above_f = scan_level(k_rem)
            eq_v[pl.ds(0, L)] = jnp.where(
                lane == 0, bin_f, jnp.where(lane == 1, above_f, 0))
            pltpu.sync_copy(eq_v.at[pl.ds(0, L)], seq_s.at[pl.ds(0, L)])
        plsc.subcore_barrier()

        pltpu.sync_copy(seq_s.at[pl.ds(0, L)], eq_v.at[pl.ds(0, L)])
        res = eq_v[pl.ds(0, L)]
        return lane0(res), lane1(res)

    with jax.named_scope("ph_lvl0"):
        zero_hist()

        @plsc.parallel_loop(0, NV, unroll=U)
        def _(i):
            off = i * L
            b = plsc.bitcast(data_v[pl.ds(off, L)], jnp.int32)
            m = (b >> 31) | jnp.int32(-2**31)
            ki = b ^ m
            data_v[pl.ds(off, L)] = plsc.bitcast(ki, jnp.float32)
            ku = plsc.bitcast(ki, jnp.uint32)
            digit = plsc.bitcast((ku >> jnp.uint32(24)) & u32_ff, jnp.int32)
            plsc.addupdate_scatter(hist_v, [(digit << 4) | lane], ones_i)

    with jax.named_scope("ph_merge0"):
        lane_merge()
        bin_f, above_f = merge_and_scan(jnp.int32(K))
    prefix = bin_f.astype(jnp.uint32)
    k_rem = jnp.int32(K) - above_f

    def level_pass(lvl, pfx, k_rem, blk_in, blk_out):
        sd = jnp.uint32(24 - 8 * lvl)
        sh = jnp.uint32(32 - 8 * lvl)
        zero_hist()

        def blk_body(bi, _):
            def hot():
                acc = zeros_i
                for j in range(BV):
                    ku = plsc.bitcast(
                        data_v[pl.ds((bi * BV + j) * L, L)], jnp.uint32)
                    match = (ku >> sh) == pfx
                    digit = plsc.bitcast((ku >> sd) & u32_ff, jnp.int32)
                    plsc.addupdate_scatter(
                        hist_v, [(digit << 4) | lane], ones_i, mask=match)
                    acc = acc + plsc.all_reduce_population_count(match)
                return acc
            if blk_in is None:
                acc = hot()
            else:
                cnt = lane0(blk_in[pl.ds(bi * L, L)])
                acc = lax.cond(cnt > 0, hot, lambda: zeros_i)
            blk_out[pl.ds(bi * L, L)] = acc
            return 0
        lax.fori_loop(0, NBC, blk_body, 0)

        lane_merge()
        bin_f, above_f = merge_and_scan(k_rem)
        return (pfx << jnp.uint32(8)) | bin_f.astype(jnp.uint32), \
            k_rem - above_f

    with jax.named_scope("ph_lvl1"):
        prefix, k_rem = level_pass(1, prefix, k_rem, None, blka_v)
    with jax.named_scope("ph_lvl23"):
        prefix, k_rem = level_pass(2, prefix, k_rem, blka_v, blkb_v)
        prefix, k_rem = level_pass(3, prefix, k_rem, blkb_v, blka_v)

    kth = prefix
    r = k_rem

    scope_mask = jax.named_scope("ph_mask")
    scope_mask.__enter__()
    other_blk0 = (1 - c) * NB
    def cnt_other(bi, acc):
        cnt = lane0(blka_v[pl.ds((other_blk0 + bi) * L, L)])
        def hot():
            a = zeros_i
            for j in range(BV):
                ku = plsc.bitcast(
                    data_v[pl.ds(((other_blk0 + bi) * BV + j) * L, L)],
                    jnp.uint32)
                a = a + plsc.all_reduce_population_count(ku == kth)
            return a
        return acc + lax.cond(cnt > 0, hot, lambda: zeros_i)
    m_other = lax.fori_loop(0, NB, cnt_other, zeros_i)

    my_blk0 = c * NB
    my_base = c * HV * L
    def mask_blk(bi, acc):
        for j in range(BV):
            moff = (bi * BV + j) * L
            ku = plsc.bitcast(data_v[pl.ds(my_base + moff, L)], jnp.uint32)
            mask_v[pl.ds(moff, L)] = jnp.where(ku > kth, 1.0, 0.0)
        cnt = lane0(blka_v[pl.ds((my_blk0 + bi) * L, L)])
        def hot():
            a = zeros_i
            for j in range(BV):
                ku = plsc.bitcast(
                    data_v[pl.ds(my_base + (bi * BV + j) * L, L)], jnp.uint32)
                a = a + plsc.all_reduce_population_count(ku == kth)
            return a
        blkeq = lax.cond(cnt > 0, hot, lambda: zeros_i)
        blk_v[pl.ds(bi * L, L)] = blkeq
        return acc + blkeq
    m_mine = lax.fori_loop(0, NB, mask_blk, zeros_i)

    c0 = jnp.where(c == 0, m_mine, m_other)
    c1 = jnp.where(c == 0, m_other, m_mine)
    eq_v[pl.ds(0, L)] = jnp.where(lane == 0, c0,
                                  jnp.where(lane == 1, c1, zeros_i))
    pltpu.sync_copy(eq_v.at[pl.ds(0, L)], seq_s.at[pl.ds(s * L, L)])
    plsc.subcore_barrier()
    pltpu.sync_copy(seq_s, eq_v)

    def base_sum(t, bacc):
        rowt = eq_v[pl.ds(t * L, L)]
        bacc = bacc + jnp.where(t < s, lane0(rowt) + lane1(rowt), 0)
        return bacc + jnp.where((t == s) & (c == 1), lane0(rowt), 0)
    rank0 = lax.fori_loop(0, NT, base_sum, jnp.int32(0))

    def fixup(bi, carry):
        cnt = lane0(blk_v[pl.ds(bi * L, L)])
        @pl.when(cnt > 0)
        def _():
            cl = carry
            for j in range(BV):
                moff = (bi * BV + j) * L
                ku = plsc.bitcast(data_v[pl.ds(my_base + moff, L)],
                                  jnp.uint32)
                eq = ku == kth
                eqi = eq.astype(jnp.int32)
                ic = plsc.cumsum(eqi)
                sel = eq & ((ic - eqi + cl) < r)
                mask_v[pl.ds(moff, L)] = jnp.where(
                    sel, 1.0, mask_v[pl.ds(moff, L)])
                cl = cl + jnp.max(ic)
        return carry + cnt
    lax.fori_loop(0, NB, fixup, rank0)
    scope_mask.__exit__(None, None, None)

    @pl.when((s != NT - 1) | (c == 0))
    def _():
        pltpu.sync_copy(mask_v, out_hbm.at[pl.ds(s * CH + c * H, H)])

    @pl.when((s == NT - 1) & (c == 1))
    def _():
        pltpu.sync_copy(mask_v.at[pl.ds(0, LAST_STORE)],
                        out_hbm.at[pl.ds(s * CH + H, LAST_STORE)])


@functools.partial(
    pl.kernel,
    out_type=jax.ShapeDtypeStruct((N,), jnp.float32),
    mesh=plsc.VectorSubcoreMesh(core_axis_name="c", subcore_axis_name="s"),
    compiler_params=pltpu.CompilerParams(needs_layout_passes=False),
    scratch_types=[
        pltpu.VMEM((CH,), jnp.float32),
        pltpu.VMEM((H,), jnp.float32),
        pltpu.VMEM((NT * 256,), jnp.int32),
        pltpu.VMEM((256,), jnp.int32),
        pltpu.VMEM((NT * L,), jnp.int32),
        pltpu.VMEM((NB * L,), jnp.int32),
        pltpu.VMEM((NBC * L,), jnp.int32),
        pltpu.VMEM((NBC * L,), jnp.int32),
        pltpu.VMEM_SHARED((NT * 256,), jnp.int32),
        pltpu.VMEM_SHARED((NT * L,), jnp.int32),
    ],
)
def _sc_topk_mask(x_hbm, out_hbm, data_v, mask_v, hist_v, merged_v, eq_v,
                  blk_v, blka_v, blkb_v, shist_s, seq_s):
    _sc_body(x_hbm, out_hbm, data_v, mask_v, hist_v, merged_v, eq_v, blk_v,
             blka_v, blkb_v, shist_s, seq_s)


def kernel(mask_logits):
    return _sc_topk_mask(mask_logits)


# speedup vs baseline: 2.0811x; 2.0811x over previous
import functools

import jax
import jax.numpy as jnp
from jax import lax
from jax.experimental import pallas as pl
from jax.experimental.pallas import tpu as pltpu
from jax.experimental.pallas import tpu_sc as plsc

N = 1_000_000
K = 100_000
L = 16
NT = 16
CH = 62_720
NV = CH // L
HV = NV // 2
H = HV * L
U = 8
TAIL = NT * CH - N
LAST_LOAD = CH - TAIL
LAST_STORE = H - TAIL
BF = 40
NBF = HV // BF


def _iota():
    return lax.iota(jnp.int32, L)


def _sc_body(x_hbm, out_hbm, data_v, mask_v, hist_v, merged_v, eq_v, blk_v,
             shist_s, seq_s):
    c = lax.axis_index("c")
    s = lax.axis_index("s")
    lane = _iota()
    ones_i = jnp.ones((L,), jnp.int32)
    zeros_i = jnp.zeros((L,), jnp.int32)
    u32_ff = jnp.uint32(0xFF)

    def lane0(v):
        return jnp.sum(jnp.where(lane == 0, v, 0))

    def lane1(v):
        return jnp.sum(jnp.where(lane == 1, v, 0))

    with jax.named_scope("ph_load"):
        @pl.when(s != NT - 1)
        def _():
            pltpu.sync_copy(x_hbm.at[pl.ds(s * CH, CH)], data_v)

        @pl.when(s == NT - 1)
        def _():
            pltpu.sync_copy(x_hbm.at[pl.ds(s * CH, LAST_LOAD)],
                            data_v.at[pl.ds(0, LAST_LOAD)])
            ninf = jnp.full((L,), -jnp.inf, jnp.float32)
            def fill(i, _):
                data_v[pl.ds(LAST_LOAD + i * L, L)] = ninf
                return 0
            lax.fori_loop(0, TAIL // L, fill, 0)

    def zero_hist():
        def z(i, _):
            base = i * (U * L)
            for j in range(U):
                hist_v[pl.ds(base + j * L, L)] = zeros_i
            return 0
        lax.fori_loop(0, 256 // U, z, 0)

    def lane_merge():
        def m(g, _):
            base = (g * L + lane) * L
            acc = zeros_i
            for l in range(L):
                acc = acc + plsc.load_gather(hist_v, [base + l])
            merged_v[pl.ds(g * L, L)] = acc
            return 0
        lax.fori_loop(0, L, m, 0)

    def scan_level(k_rem):
        def scan_g(gi, carry):
            acc, found, bin_f, above_f = carry
            gd = 15 - gi
            def gsum(j, a):
                return a + hist_v[pl.ds(j * 256 + gd * L, L)]
            tot = lax.fori_loop(0, NT, gsum, zeros_i)
            tr = jnp.flip(tot, 0)
            cs = plsc.cumsum(tr)
            sfx = cs + acc
            hit = sfx >= k_rem
            anyh = jnp.any(hit)
            p = jnp.max(plsc.all_reduce_ffs(hit))
            onehot = lane == p
            sfx_at = jnp.sum(jnp.where(onehot, sfx, 0))
            h_at = jnp.sum(jnp.where(onehot, tr, 0))
            new = anyh & (found == 0)
            bin_f = jnp.where(new, gd * L + 15 - p, bin_f)
            above_f = jnp.where(new, sfx_at - h_at, above_f)
            found = jnp.where(anyh, 1, found)
            acc = acc + jnp.max(cs)
            return acc, found, bin_f, above_f
        _, _, bin_f, above_f = lax.fori_loop(
            0, L, scan_g,
            (jnp.int32(0), jnp.int32(0), jnp.int32(0), jnp.int32(0)))
        return bin_f, above_f

    def merge_and_scan(k_rem):
        pltpu.sync_copy(merged_v, shist_s.at[pl.ds(s * 256, 256)])
        plsc.subcore_barrier()

        @pl.when(s == 0)
        def _():
            pltpu.sync_copy(shist_s, hist_v)
            bin_f, above_f = scan_level(k_rem)
            eq_v[pl.ds(0, L)] = jnp.where(
                lane == 0, bin_f, jnp.where(lane == 1, above_f, 0))
            pltpu.sync_copy(eq_v.at[pl.ds(0, L)], seq_s.at[pl.ds(0, L)])
        plsc.subcore_barrier()

        pltpu.sync_copy(seq_s.at[pl.ds(0, L)], eq_v.at[pl.ds(0, L)])
        res = eq_v[pl.ds(0, L)]
        return lane0(res), lane1(res)

    with jax.named_scope("ph_lvl0"):
        zero_hist()

        @plsc.parallel_loop(0, NV, unroll=U)
        def _(i):
            off = i * L
            b = plsc.bitcast(data_v[pl.ds(off, L)], jnp.int32)
            m = (b >> 31) | jnp.int32(-2**31)
            ki = b ^ m
            data_v[pl.ds(off, L)] = plsc.bitcast(ki, jnp.float32)
            ku = plsc.bitcast(ki, jnp.uint32)
            digit = plsc.bitcast((ku >> jnp.uint32(24)) & u32_ff, jnp.int32)
            plsc.addupdate_scatter(hist_v, [(digit << 4) | lane], ones_i)

    with jax.named_scope("ph_merge0"):
        lane_merge()
        bin_f, above_f = merge_and_scan(jnp.int32(K))
    prefix = bin_f.astype(jnp.uint32)
    k_rem = jnp.int32(K) - above_f

    def level_pass(lvl, pfx, k_rem):
        sd = jnp.uint32(24 - 8 * lvl)
        sh = jnp.uint32(32 - 8 * lvl)
        zero_hist()

        @plsc.parallel_loop(0, NV, unroll=U)
        def _(i):
            ku = plsc.bitcast(data_v[pl.ds(i * L, L)], jnp.uint32)
            match = (ku >> sh) == pfx
            digit = plsc.bitcast((ku >> sd) & u32_ff, jnp.int32)
            plsc.addupdate_scatter(
                hist_v, [(digit << 4) | lane], ones_i, mask=match)

        lane_merge()
        bin_f, above_f = merge_and_scan(k_rem)
        return (pfx << jnp.uint32(8)) | bin_f.astype(jnp.uint32), \
            k_rem - above_f

    with jax.named_scope("ph_lvl1"):
        prefix, k_rem = level_pass(1, prefix, k_rem)
    with jax.named_scope("ph_lvl23"):
        prefix, k_rem = level_pass(2, prefix, k_rem)
        prefix, k_rem = level_pass(3, prefix, k_rem)

    kth = prefix
    r = k_rem

    scope_mask = jax.named_scope("ph_mask")
    scope_mask.__enter__()
    other_base = (1 - c) * HV * L

    @plsc.parallel_loop(0, HV, unroll=U, carry=jnp.zeros((L,), jnp.int32))
    def m_other(i, acc):
        ku = plsc.bitcast(data_v[pl.ds(other_base + i * L, L)], jnp.uint32)
        return acc + plsc.all_reduce_population_count(ku == kth)

    my_base = c * HV * L
    def mask_blk(bi, acc):
        blkeq = zeros_i
        for j in range(BF):
            moff = (bi * BF + j) * L
            ku = plsc.bitcast(data_v[pl.ds(my_base + moff, L)], jnp.uint32)
            mask_v[pl.ds(moff, L)] = jnp.where(ku > kth, 1.0, 0.0)
            blkeq = blkeq + plsc.all_reduce_population_count(ku == kth)
        blk_v[pl.ds(bi * L, L)] = blkeq
        return acc + blkeq
    m_mine = lax.fori_loop(0, NBF, mask_blk, zeros_i)

    c0 = jnp.where(c == 0, m_mine, m_other)
    c1 = jnp.where(c == 0, m_other, m_mine)
    eq_v[pl.ds(0, L)] = jnp.where(lane == 0, c0,
                                  jnp.where(lane == 1, c1, zeros_i))
    pltpu.sync_copy(eq_v.at[pl.ds(0, L)], seq_s.at[pl.ds(s * L, L)])
    plsc.subcore_barrier()
    pltpu.sync_copy(seq_s, eq_v)

    def base_sum(t, bacc):
        rowt = eq_v[pl.ds(t * L, L)]
        bacc = bacc + jnp.where(t < s, lane0(rowt) + lane1(rowt), 0)
        return bacc + jnp.where((t == s) & (c == 1), lane0(rowt), 0)
    rank0 = lax.fori_loop(0, NT, base_sum, jnp.int32(0))

    def fixup(bi, carry):
        cnt = lane0(blk_v[pl.ds(bi * L, L)])
        @pl.when(cnt > 0)
        def _():
            cl = carry
            for j in range(BF):
                moff = (bi * BF + j) * L
                ku = plsc.bitcast(data_v[pl.ds(my_base + moff, L)],
                                  jnp.uint32)
                eq = ku == kth
                eqi = eq.astype(jnp.int32)
                ic = plsc.cumsum(eqi)
                sel = eq & ((ic - eqi + cl) < r)
                mask_v[pl.ds(moff, L)] = jnp.where(
                    sel, 1.0, mask_v[pl.ds(moff, L)])
                cl = cl + jnp.max(ic)
        return carry + cnt
    lax.fori_loop(0, NBF, fixup, rank0)
    scope_mask.__exit__(None, None, None)

    @pl.when((s != NT - 1) | (c == 0))
    def _():
        pltpu.sync_copy(mask_v, out_hbm.at[pl.ds(s * CH + c * H, H)])

    @pl.when((s == NT - 1) & (c == 1))
    def _():
        pltpu.sync_copy(mask_v.at[pl.ds(0, LAST_STORE)],
                        out_hbm.at[pl.ds(s * CH + H, LAST_STORE)])


@functools.partial(
    pl.kernel,
    out_type=jax.ShapeDtypeStruct((N,), jnp.float32),
    mesh=plsc.VectorSubcoreMesh(core_axis_name="c", subcore_axis_name="s"),
    compiler_params=pltpu.CompilerParams(needs_layout_passes=False),
    scratch_types=[
        pltpu.VMEM((CH,), jnp.float32),
        pltpu.VMEM((H,), jnp.float32),
        pltpu.VMEM((NT * 256,), jnp.int32),
        pltpu.VMEM((256,), jnp.int32),
        pltpu.VMEM((NT * L,), jnp.int32),
        pltpu.VMEM((NBF * L,), jnp.int32),
        pltpu.VMEM_SHARED((NT * 256,), jnp.int32),
        pltpu.VMEM_SHARED((NT * L,), jnp.int32),
    ],
)
def _sc_topk_mask(x_hbm, out_hbm, data_v, mask_v, hist_v, merged_v, eq_v,
                  blk_v, shist_s, seq_s):
    _sc_body(x_hbm, out_hbm, data_v, mask_v, hist_v, merged_v, eq_v, blk_v,
             shist_s, seq_s)


def kernel(mask_logits):
    return _sc_topk_mask(mask_logits)
